# baseline (speedup 1.0000x reference)
import functools

import jax
import jax.numpy as jnp
from jax import lax
from jax.experimental import pallas as pl
from jax.experimental.pallas import tpu as pltpu
from jax.experimental.pallas import tpu_sc as plsc

LEN = 200
BATCH = 1024
DIM = 64
OUT_DIM = 64
B = LEN * BATCH
VOCAB_PAIRS = 500000

NC = 2
NS = 16
NW = NC * NS
B_PER_W = B // NW
GCHUNK = 128
N_GCHUNKS = B_PER_W // GCHUNK
NBUF = 2
NGROUP = N_GCHUNKS // NBUF


@functools.cache
def _make_sc_gather():
    mesh = plsc.VectorSubcoreMesh(core_axis_name="c", subcore_axis_name="s")

    @functools.partial(
        pl.kernel,
        mesh=mesh,
        compiler_params=pltpu.CompilerParams(use_tc_tiling_on_sc=True),
        out_type=jax.ShapeDtypeStruct((B, 2 * DIM), jnp.float32),
        scratch_types=[
            pltpu.VMEM((N_GCHUNKS, GCHUNK), jnp.int32),
            pltpu.VMEM((2, NBUF, GCHUNK, 2 * DIM), jnp.float32),
            pltpu.SemaphoreType.DMA,
            pltpu.SemaphoreType.DMA,
        ],
    )
    def _sc_gather(idx_hbm, table_hbm, out_hbm, idx_v, rows_v, gsem, wsem):
        wid = lax.axis_index("s") * NC + lax.axis_index("c")
        base = wid * B_PER_W
        pltpu.sync_copy(idx_hbm.at[wid], idx_v)

        def group(g, carry):
            p = lax.rem(g, 2)

            @pl.when(g >= 2)
            def _():
                for b in range(NBUF):
                    pltpu.make_async_copy(
                        rows_v.at[p, b], out_hbm.at[pl.ds(base, GCHUNK)], wsem
                    ).wait()

            descs = []
            for b in range(NBUF):
                j = g * NBUF + b
                descs.append(
                    pltpu.async_copy(table_hbm.at[idx_v.at[j]], rows_v.at[p, b], gsem)
                )
            for d in descs:
                d.wait()
            for b in range(NBUF):
                j = g * NBUF + b
                pltpu.async_copy(
                    rows_v.at[p, b], out_hbm.at[pl.ds(base + j * GCHUNK, GCHUNK)], wsem
                )
            return carry

        lax.fori_loop(0, NGROUP, group, 0)
        for p in range(2):
            for b in range(NBUF):
                pltpu.make_async_copy(
                    rows_v.at[p, b], out_hbm.at[pl.ds(base, GCHUNK)], wsem
                ).wait()

    return _sc_gather


ROWS_BLK = 2048


def _proj_body(x_ref, h_ref, w_ref, b_ref, o_ref):
    xl = x_ref[:, :DIM]
    xr = x_ref[:, DIM:]
    x = jnp.where(h_ref[...] == 1, xr, xl)
    acc = jnp.dot(x, w_ref[...], preferred_element_type=jnp.float32)
    o_ref[...] = jnp.maximum(acc + b_ref[...], 0.0)


def _project(g, h, W_proj, b_proj):
    return pl.pallas_call(
        _proj_body,
        grid=(B // ROWS_BLK,),
        in_specs=[
            pl.BlockSpec((ROWS_BLK, 2 * DIM), lambda i: (i, 0)),
            pl.BlockSpec((ROWS_BLK, 1), lambda i: (i, 0)),
            pl.BlockSpec((DIM, OUT_DIM), lambda i: (0, 0)),
            pl.BlockSpec((1, OUT_DIM), lambda i: (0, 0)),
        ],
        out_specs=pl.BlockSpec((ROWS_BLK, OUT_DIM), lambda i: (i, 0)),
        out_shape=jax.ShapeDtypeStruct((B, OUT_DIM), jnp.float32),
    )(g, h, W_proj, b_proj.reshape(1, OUT_DIM))


def kernel(input, W_emb, W_proj, b_proj):
    idx = input.reshape(B).astype(jnp.int32)
    idx_pair = (idx >> 1).reshape(NW, N_GCHUNKS, GCHUNK)
    half = (idx & 1).reshape(B, 1)
    table = W_emb.reshape(VOCAB_PAIRS, 2 * DIM)
    g = _make_sc_gather()(idx_pair, table)
    out = _project(g, half, W_proj, b_proj)
    return out.reshape(LEN, BATCH, OUT_DIM)


# speedup vs baseline: 1.4447x; 1.4447x over previous
import functools

import jax
import jax.numpy as jnp
from jax import lax
from jax.experimental import pallas as pl
from jax.experimental.pallas import tpu as pltpu
from jax.experimental.pallas import tpu_sc as plsc

LEN = 200
BATCH = 1024
DIM = 64
OUT_DIM = 64
B = LEN * BATCH
VOCAB = 1000000

NC = 2
NS = 16
NW = NC * NS
B_PER_W = B // NW
GCHUNK = 128
N_GCHUNKS = B_PER_W // GCHUNK
NBUF = 5
NGROUP = N_GCHUNKS // NBUF

VB = 2048
NVB = -(-VOCAB // VB)
VOCAB_PAD = NVB * VB


def _pack_body(xt_ref, w_ref, b_ref, o_ref):
    acc = jax.lax.dot_general(
        xt_ref[...], w_ref[...],
        dimension_numbers=(((0,), (0,)), ((), ())),
        preferred_element_type=jnp.float32,
    )
    p = jnp.maximum(acc + b_ref[...], 0.0)
    o_ref[:, :OUT_DIM] = p[: VB // 2]
    o_ref[:, OUT_DIM:] = p[VB // 2 :]


def _pack_project(Wt, W_proj, b_proj):
    return pl.pallas_call(
        _pack_body,
        grid=(NVB,),
        in_specs=[
            pl.BlockSpec((DIM, VB), lambda i: (0, i)),
            pl.BlockSpec((DIM, OUT_DIM), lambda i: (0, 0)),
            pl.BlockSpec((1, OUT_DIM), lambda i: (0, 0)),
        ],
        out_specs=pl.BlockSpec((VB // 2, 2 * OUT_DIM), lambda i: (i, 0)),
        out_shape=jax.ShapeDtypeStruct((VOCAB_PAD // 2, 2 * OUT_DIM), jnp.float32),
    )(Wt, W_proj, b_proj.reshape(1, OUT_DIM))


@functools.cache
def _make_sc_gather():
    mesh = plsc.VectorSubcoreMesh(core_axis_name="c", subcore_axis_name="s")

    @functools.partial(
        pl.kernel,
        mesh=mesh,
        compiler_params=pltpu.CompilerParams(use_tc_tiling_on_sc=False),
        out_type=jax.ShapeDtypeStruct((B, DIM), jnp.float32),
        scratch_types=[
            pltpu.VMEM((N_GCHUNKS, GCHUNK), jnp.int32),
            pltpu.VMEM((2, NBUF, GCHUNK, DIM), jnp.float32),
            pltpu.SemaphoreType.DMA,
            pltpu.SemaphoreType.DMA,
        ],
    )
    def _sc_gather(idx_hbm, table_hbm, out_hbm, idx_v, rows_v, gsem, wsem):
        wid = lax.axis_index("s") * NC + lax.axis_index("c")
        base = wid * B_PER_W
        pltpu.sync_copy(idx_hbm.at[wid], idx_v)

        def group(g, carry):
            p = lax.rem(g, 2)

            @pl.when(g >= 2)
            def _():
                for b in range(NBUF):
                    pltpu.make_async_copy(
                        rows_v.at[p, b], out_hbm.at[pl.ds(base, GCHUNK)], wsem
                    ).wait()

            descs = []
            for b in range(NBUF):
                j = g * NBUF + b
                descs.append(
                    pltpu.async_copy(table_hbm.at[idx_v.at[j]], rows_v.at[p, b], gsem)
                )
            for d in descs:
                d.wait()
            for b in range(NBUF):
                j = g * NBUF + b
                pltpu.async_copy(
                    rows_v.at[p, b], out_hbm.at[pl.ds(base + j * GCHUNK, GCHUNK)], wsem
                )
            return carry

        lax.fori_loop(0, NGROUP, group, 0)
        for p in range(2):
            for b in range(NBUF):
                pltpu.make_async_copy(
                    rows_v.at[p, b], out_hbm.at[pl.ds(base, GCHUNK)], wsem
                ).wait()

    return _sc_gather


def kernel(input, W_emb, W_proj, b_proj):
    idx = input.reshape(B).astype(jnp.int32)
    j = idx & (VB - 1)
    hi = j >> 10
    u = ((idx >> 11) << 11) + ((j & (VB // 2 - 1)) << 1) + hi
    u = u.reshape(NW, N_GCHUNKS, GCHUNK)
    P2 = _pack_project(W_emb.T, W_proj, b_proj)
    P = P2.reshape(VOCAB_PAD, DIM)
    out = _make_sc_gather()(u, P)
    return out.reshape(LEN, BATCH, OUT_DIM)


# speedup vs baseline: 1.5282x; 1.0578x over previous
import functools

import jax
import jax.numpy as jnp
from jax import lax
from jax.experimental import pallas as pl
from jax.experimental.pallas import tpu as pltpu
from jax.experimental.pallas import tpu_sc as plsc

LEN = 200
BATCH = 1024
DIM = 64
OUT_DIM = 64
B = LEN * BATCH
VOCAB = 1000000

NC = 2
NS = 16
NW = NC * NS
B_PER_W = B // NW
GCHUNK = 128
N_GCHUNKS = B_PER_W // GCHUNK
NBUF = 5
NGROUP = N_GCHUNKS // NBUF

VB = 2048
NVB = -(-VOCAB // VB)
VOCAB_PAD = NVB * VB


def _pack_body(xt_ref, w_ref, b_ref, o_ref):
    acc = jax.lax.dot_general(
        xt_ref[...].astype(jnp.bfloat16), w_ref[...].astype(jnp.bfloat16),
        dimension_numbers=(((0,), (0,)), ((), ())),
        preferred_element_type=jnp.float32,
    )
    p = jnp.maximum(acc + b_ref[...], 0.0)
    o_ref[:, :OUT_DIM] = p[: VB // 2]
    o_ref[:, OUT_DIM:] = p[VB // 2 :]


def _pack_project(Wt, W_proj, b_proj):
    return pl.pallas_call(
        _pack_body,
        grid=(NVB,),
        in_specs=[
            pl.BlockSpec((DIM, VB), lambda i: (0, i)),
            pl.BlockSpec((DIM, OUT_DIM), lambda i: (0, 0)),
            pl.BlockSpec((1, OUT_DIM), lambda i: (0, 0)),
        ],
        out_specs=pl.BlockSpec((VB // 2, 2 * OUT_DIM), lambda i: (i, 0)),
        out_shape=jax.ShapeDtypeStruct((VOCAB_PAD // 2, 2 * OUT_DIM), jnp.float32),
    )(Wt, W_proj, b_proj.reshape(1, OUT_DIM))


@functools.cache
def _make_sc_gather():
    mesh = plsc.VectorSubcoreMesh(core_axis_name="c", subcore_axis_name="s")

    @functools.partial(
        pl.kernel,
        mesh=mesh,
        compiler_params=pltpu.CompilerParams(use_tc_tiling_on_sc=False),
        out_type=jax.ShapeDtypeStruct((B, DIM), jnp.float32),
        scratch_types=[
            pltpu.VMEM((N_GCHUNKS, GCHUNK), jnp.int32),
            pltpu.VMEM((2, NBUF, GCHUNK, DIM), jnp.float32),
            pltpu.SemaphoreType.DMA,
            pltpu.SemaphoreType.DMA,
        ],
    )
    def _sc_gather(idx_hbm, table_hbm, out_hbm, idx_v, rows_v, gsem, wsem):
        wid = lax.axis_index("s") * NC + lax.axis_index("c")
        base = wid * B_PER_W
        pltpu.sync_copy(idx_hbm.at[wid], idx_v)

        def group(g, carry):
            p = lax.rem(g, 2)

            @pl.when(g >= 2)
            def _():
                for b in range(NBUF):
                    pltpu.make_async_copy(
                        rows_v.at[p, b], out_hbm.at[pl.ds(base, GCHUNK)], wsem
                    ).wait()

            descs = []
            for b in range(NBUF):
                j = g * NBUF + b
                descs.append(
                    pltpu.async_copy(table_hbm.at[idx_v.at[j]], rows_v.at[p, b], gsem)
                )
            for d in descs:
                d.wait()
            for b in range(NBUF):
                j = g * NBUF + b
                pltpu.async_copy(
                    rows_v.at[p, b], out_hbm.at[pl.ds(base + j * GCHUNK, GCHUNK)], wsem
                )
            return carry

        lax.fori_loop(0, NGROUP, group, 0)
        for p in range(2):
            for b in range(NBUF):
                pltpu.make_async_copy(
                    rows_v.at[p, b], out_hbm.at[pl.ds(base, GCHUNK)], wsem
                ).wait()

    return _sc_gather


def kernel(input, W_emb, W_proj, b_proj):
    idx = input.reshape(B).astype(jnp.int32)
    j = idx & (VB - 1)
    hi = j >> 10
    u = ((idx >> 11) << 11) + ((j & (VB // 2 - 1)) << 1) + hi
    u = u.reshape(NW, N_GCHUNKS, GCHUNK)
    P2 = _pack_project(W_emb.T, W_proj, b_proj)
    P = P2.reshape(VOCAB_PAD, DIM)
    out = _make_sc_gather()(u, P)
    return out.reshape(LEN, BATCH, OUT_DIM)


# speedup vs baseline: 2.2790x; 1.4913x over previous
import functools

import jax
import jax.numpy as jnp
from jax import lax
from jax.experimental import pallas as pl
from jax.experimental.pallas import tpu as pltpu
from jax.experimental.pallas import tpu_sc as plsc

LEN = 200
BATCH = 1024
DIM = 64
OUT_DIM = 64
B = LEN * BATCH
VOCAB = 1000000

NC = 2
NS = 16
NW = NC * NS
B_PER_W = B // NW
GCHUNK = 128
N_GCHUNKS = B_PER_W // GCHUNK
NBUF = 5
NGROUP = N_GCHUNKS // NBUF

VB = 8192
NVB = -(-VOCAB // VB)
VOCAB_PAD = NVB * VB


def _pack_body(xt_ref, w_ref, b_ref, o_ref):
    acc = jax.lax.dot_general(
        xt_ref[...].astype(jnp.bfloat16), w_ref[...].astype(jnp.bfloat16),
        dimension_numbers=(((0,), (0,)), ((), ())),
        preferred_element_type=jnp.float32,
    )
    p = jnp.maximum(acc + b_ref[...], 0.0)
    o_ref[:, :OUT_DIM] = p[: VB // 2]
    o_ref[:, OUT_DIM:] = p[VB // 2 :]


def _pack_project(Wt, W_proj, b_proj):
    return pl.pallas_call(
        _pack_body,
        grid=(NVB,),
        in_specs=[
            pl.BlockSpec((DIM, VB), lambda i: (0, i)),
            pl.BlockSpec((DIM, OUT_DIM), lambda i: (0, 0)),
            pl.BlockSpec((1, OUT_DIM), lambda i: (0, 0)),
        ],
        out_specs=pl.BlockSpec((VB // 2, 2 * OUT_DIM), lambda i: (i, 0)),
        out_shape=jax.ShapeDtypeStruct((VOCAB_PAD // 2, 2 * OUT_DIM), jnp.float32),
        compiler_params=pltpu.CompilerParams(fuse_transposed_lhs_in_matmul=True),
    )(Wt, W_proj, b_proj.reshape(1, OUT_DIM))


@functools.cache
def _make_sc_gather():
    mesh = plsc.VectorSubcoreMesh(core_axis_name="c", subcore_axis_name="s")

    @functools.partial(
        pl.kernel,
        mesh=mesh,
        compiler_params=pltpu.CompilerParams(use_tc_tiling_on_sc=False),
        out_type=jax.ShapeDtypeStruct((B, DIM), jnp.float32),
        scratch_types=[
            pltpu.VMEM((N_GCHUNKS, GCHUNK), jnp.int32),
            pltpu.VMEM((2, NBUF, GCHUNK, DIM), jnp.float32),
            pltpu.SemaphoreType.DMA,
            pltpu.SemaphoreType.DMA,
        ],
    )
    def _sc_gather(idx_hbm, table_hbm, out_hbm, idx_v, rows_v, gsem, wsem):
        wid = lax.axis_index("s") * NC + lax.axis_index("c")
        base = wid * B_PER_W
        pltpu.sync_copy(idx_hbm.at[wid], idx_v)

        def group(g, carry):
            p = lax.rem(g, 2)

            @pl.when(g >= 2)
            def _():
                for b in range(NBUF):
                    pltpu.make_async_copy(
                        rows_v.at[p, b], out_hbm.at[pl.ds(base, GCHUNK)], wsem
                    ).wait()

            descs = []
            for b in range(NBUF):
                j = g * NBUF + b
                descs.append(
                    pltpu.async_copy(table_hbm.at[idx_v.at[j]], rows_v.at[p, b], gsem)
                )
            for d in descs:
                d.wait()
            for b in range(NBUF):
                j = g * NBUF + b
                pltpu.async_copy(
                    rows_v.at[p, b], out_hbm.at[pl.ds(base + j * GCHUNK, GCHUNK)], wsem
                )
            return carry

        lax.fori_loop(0, NGROUP, group, 0)
        for p in range(2):
            for b in range(NBUF):
                pltpu.make_async_copy(
                    rows_v.at[p, b], out_hbm.at[pl.ds(base, GCHUNK)], wsem
                ).wait()

    return _sc_gather


def kernel(input, W_emb, W_proj, b_proj):
    idx = input.reshape(B).astype(jnp.int32)
    sh = VB.bit_length() - 1
    j = idx & (VB - 1)
    hi = j >> (sh - 1)
    u = ((idx >> sh) << sh) + ((j & (VB // 2 - 1)) << 1) + hi
    u = u.reshape(NW, N_GCHUNKS, GCHUNK)
    P2 = _pack_project(W_emb.T, W_proj, b_proj)
    P = P2.reshape(VOCAB_PAD, DIM)
    out = _make_sc_gather()(u, P)
    return out.reshape(LEN, BATCH, OUT_DIM)


# speedup vs baseline: 2.4963x; 1.0954x over previous
import functools

import jax
import jax.numpy as jnp
from jax import lax
from jax.experimental import pallas as pl
from jax.experimental.pallas import tpu as pltpu
from jax.experimental.pallas import tpu_sc as plsc

LEN = 200
BATCH = 1024
DIM = 64
OUT_DIM = 64
B = LEN * BATCH
VOCAB = 1000000

NC = 2
NS = 16
NW = NC * NS
B_PER_W = B // NW
GCHUNK = 128
N_GCHUNKS = B_PER_W // GCHUNK
NBUF = 5
NGROUP = N_GCHUNKS // NBUF

VB = 16384
NVB = -(-VOCAB // VB)
VOCAB_PAD = NVB * VB


def _pack_body(xt_ref, w_ref, b_ref, o_ref):
    acc = jax.lax.dot_general(
        xt_ref[...].astype(jnp.bfloat16), w_ref[...].astype(jnp.bfloat16),
        dimension_numbers=(((0,), (0,)), ((), ())),
        preferred_element_type=jnp.float32,
    )
    p = jnp.maximum(acc + b_ref[...], 0.0)
    o_ref[:, :OUT_DIM] = p[: VB // 2]
    o_ref[:, OUT_DIM:] = p[VB // 2 :]


def _pack_project(Wt, W_proj, b_proj):
    return pl.pallas_call(
        _pack_body,
        grid=(NVB,),
        in_specs=[
            pl.BlockSpec((DIM, VB), lambda i: (0, i)),
            pl.BlockSpec((DIM, OUT_DIM), lambda i: (0, 0)),
            pl.BlockSpec((1, OUT_DIM), lambda i: (0, 0)),
        ],
        out_specs=pl.BlockSpec((VB // 2, 2 * OUT_DIM), lambda i: (i, 0)),
        out_shape=jax.ShapeDtypeStruct((VOCAB_PAD // 2, 2 * OUT_DIM), jnp.float32),
        compiler_params=pltpu.CompilerParams(fuse_transposed_lhs_in_matmul=True),
    )(Wt, W_proj, b_proj.reshape(1, OUT_DIM))


@functools.cache
def _make_sc_gather():
    mesh = plsc.VectorSubcoreMesh(core_axis_name="c", subcore_axis_name="s")

    @functools.partial(
        pl.kernel,
        mesh=mesh,
        compiler_params=pltpu.CompilerParams(use_tc_tiling_on_sc=False),
        out_type=jax.ShapeDtypeStruct((B, DIM), jnp.float32),
        scratch_types=[
            pltpu.VMEM((N_GCHUNKS, GCHUNK), jnp.int32),
            pltpu.VMEM((2, NBUF, GCHUNK, DIM), jnp.float32),
            pltpu.SemaphoreType.DMA,
            pltpu.SemaphoreType.DMA,
        ],
    )
    def _sc_gather(idx_hbm, table_hbm, out_hbm, idx_v, rows_v, gsem, wsem):
        wid = lax.axis_index("s") * NC + lax.axis_index("c")
        base = wid * B_PER_W
        pltpu.sync_copy(idx_hbm.at[wid], idx_v)

        def group(g, carry):
            p = lax.rem(g, 2)

            @pl.when(g >= 2)
            def _():
                for b in range(NBUF):
                    pltpu.make_async_copy(
                        rows_v.at[p, b], out_hbm.at[pl.ds(base, GCHUNK)], wsem
                    ).wait()

            descs = []
            for b in range(NBUF):
                j = g * NBUF + b
                descs.append(
                    pltpu.async_copy(table_hbm.at[idx_v.at[j]], rows_v.at[p, b], gsem)
                )
            for d in descs:
                d.wait()
            for b in range(NBUF):
                j = g * NBUF + b
                pltpu.async_copy(
                    rows_v.at[p, b], out_hbm.at[pl.ds(base + j * GCHUNK, GCHUNK)], wsem
                )
            return carry

        lax.fori_loop(0, NGROUP, group, 0)
        for p in range(2):
            for b in range(NBUF):
                pltpu.make_async_copy(
                    rows_v.at[p, b], out_hbm.at[pl.ds(base, GCHUNK)], wsem
                ).wait()

    return _sc_gather


def kernel(input, W_emb, W_proj, b_proj):
    idx = input.reshape(B).astype(jnp.int32)
    sh = VB.bit_length() - 1
    j = idx & (VB - 1)
    hi = j >> (sh - 1)
    u = ((idx >> sh) << sh) + ((j & (VB // 2 - 1)) << 1) + hi
    u = u.reshape(NW, N_GCHUNKS, GCHUNK)
    P2 = _pack_project(W_emb.T, W_proj, b_proj)
    P = P2.reshape(VOCAB_PAD, DIM)
    out = _make_sc_gather()(u, P)
    return out.reshape(LEN, BATCH, OUT_DIM)


# speedup vs baseline: 2.6330x; 1.0548x over previous
import functools

import jax
import jax.numpy as jnp
from jax import lax
from jax.experimental import pallas as pl
from jax.experimental.pallas import tpu as pltpu
from jax.experimental.pallas import tpu_sc as plsc

LEN = 200
BATCH = 1024
DIM = 64
OUT_DIM = 64
B = LEN * BATCH
VOCAB = 1000000

NC = 2
NS = 16
NW = NC * NS
B_PER_W = B // NW
GCHUNK = 128
N_GCHUNKS = B_PER_W // GCHUNK
NBUF = 5
NGROUP = N_GCHUNKS // NBUF

VB = 32768
NVB = -(-VOCAB // VB)
VOCAB_PAD = NVB * VB


def _pack_body(xt_ref, w_ref, b_ref, o_ref):
    acc = jax.lax.dot_general(
        xt_ref[...].astype(jnp.bfloat16), w_ref[...].astype(jnp.bfloat16),
        dimension_numbers=(((0,), (0,)), ((), ())),
        preferred_element_type=jnp.float32,
    )
    p = jnp.maximum(acc + b_ref[...], 0.0)
    o_ref[:, :OUT_DIM] = p[: VB // 2]
    o_ref[:, OUT_DIM:] = p[VB // 2 :]


def _pack_project(Wt, W_proj, b_proj):
    return pl.pallas_call(
        _pack_body,
        grid=(NVB,),
        in_specs=[
            pl.BlockSpec((DIM, VB), lambda i: (0, i)),
            pl.BlockSpec((DIM, OUT_DIM), lambda i: (0, 0)),
            pl.BlockSpec((1, OUT_DIM), lambda i: (0, 0)),
        ],
        out_specs=pl.BlockSpec((VB // 2, 2 * OUT_DIM), lambda i: (i, 0)),
        out_shape=jax.ShapeDtypeStruct((VOCAB_PAD // 2, 2 * OUT_DIM), jnp.float32),
        compiler_params=pltpu.CompilerParams(fuse_transposed_lhs_in_matmul=True),
    )(Wt, W_proj, b_proj.reshape(1, OUT_DIM))


@functools.cache
def _make_sc_gather():
    mesh = plsc.VectorSubcoreMesh(core_axis_name="c", subcore_axis_name="s")

    @functools.partial(
        pl.kernel,
        mesh=mesh,
        compiler_params=pltpu.CompilerParams(use_tc_tiling_on_sc=False),
        out_type=jax.ShapeDtypeStruct((B, DIM), jnp.float32),
        scratch_types=[
            pltpu.VMEM((N_GCHUNKS, GCHUNK), jnp.int32),
            pltpu.VMEM((2, NBUF, GCHUNK, DIM), jnp.float32),
            pltpu.SemaphoreType.DMA,
            pltpu.SemaphoreType.DMA,
        ],
    )
    def _sc_gather(idx_hbm, table_hbm, out_hbm, idx_v, rows_v, gsem, wsem):
        wid = lax.axis_index("s") * NC + lax.axis_index("c")
        base = wid * B_PER_W
        pltpu.sync_copy(idx_hbm.at[wid], idx_v)

        def group(g, carry):
            p = lax.rem(g, 2)

            @pl.when(g >= 2)
            def _():
                for b in range(NBUF):
                    pltpu.make_async_copy(
                        rows_v.at[p, b], out_hbm.at[pl.ds(base, GCHUNK)], wsem
                    ).wait()

            descs = []
            for b in range(NBUF):
                j = g * NBUF + b
                descs.append(
                    pltpu.async_copy(table_hbm.at[idx_v.at[j]], rows_v.at[p, b], gsem)
                )
            for d in descs:
                d.wait()
            for b in range(NBUF):
                j = g * NBUF + b
                pltpu.async_copy(
                    rows_v.at[p, b], out_hbm.at[pl.ds(base + j * GCHUNK, GCHUNK)], wsem
                )
            return carry

        lax.fori_loop(0, NGROUP, group, 0)
        for p in range(2):
            for b in range(NBUF):
                pltpu.make_async_copy(
                    rows_v.at[p, b], out_hbm.at[pl.ds(base, GCHUNK)], wsem
                ).wait()

    return _sc_gather


def kernel(input, W_emb, W_proj, b_proj):
    idx = input.reshape(B).astype(jnp.int32)
    sh = VB.bit_length() - 1
    j = idx & (VB - 1)
    hi = j >> (sh - 1)
    u = ((idx >> sh) << sh) + ((j & (VB // 2 - 1)) << 1) + hi
    u = u.reshape(NW, N_GCHUNKS, GCHUNK)
    P2 = _pack_project(W_emb.T, W_proj, b_proj)
    P = P2.reshape(VOCAB_PAD, DIM)
    out = _make_sc_gather()(u, P)
    return out.reshape(LEN, BATCH, OUT_DIM)
